# initial kernel scaffold (unmeasured)
import jax
import jax.numpy as jnp
from jax import lax
from jax.experimental import pallas as pl
from jax.experimental.pallas import tpu as pltpu

T = 2048
D = 4096
V_SHARD = 8192
TV = 512
NV = V_SHARD // TV


def kernel(x, W, labels):
    x = x.astype(jnp.bfloat16)
    labels2d = labels.reshape(T, 1)

    def body(x_ref, w_ref, lab_ref, out_ref,
             acc_ref, recv_ref, send_sem, recv_sem):
        j = pl.program_id(0)

        @pl.when(j == 0)
        def _init():
            acc_ref[...] = jnp.zeros_like(acc_ref)

        logits = jnp.dot(
            x_ref[...], w_ref[...].astype(jnp.bfloat16),
            preferred_element_type=jnp.float32,
        )

        s_part = jnp.sum(jnp.exp(logits), axis=1, keepdims=True)

        my_x = lax.axis_index("x")
        col0 = my_x * V_SHARD + j * TV
        cols = col0 + lax.broadcasted_iota(jnp.int32, (1, TV), 1)
        mask = lab_ref[...] == cols
        ll_part = jnp.sum(jnp.where(mask, logits, 0.0), axis=1, keepdims=True)

        acc_ref[:, 0:1] += s_part
        acc_ref[:, 1:2] += ll_part

        @pl.when(j == NV - 1)
        def _finish():
            my_y = lax.axis_index("y")
            my_z = lax.axis_index("z")
            rdma = pltpu.make_async_remote_copy(
                src_ref=acc_ref,
                dst_ref=recv_ref,
                send_sem=send_sem,
                recv_sem=recv_sem,
                device_id=(1 - my_x, my_y, my_z),
                device_id_type=pl.DeviceIdType.MESH,
            )
            rdma.start()
            rdma.wait()
            s_tot = acc_ref[:, 0:1] + recv_ref[:, 0:1]
            ll_tot = acc_ref[:, 1:2] + recv_ref[:, 1:2]
            out_ref[...] = jnp.log(s_tot) - ll_tot

    out = pl.pallas_call(
        body,
        grid=(NV,),
        out_shape=jax.ShapeDtypeStruct((T, 1), jnp.float32),
        in_specs=[
            pl.BlockSpec((T, D), lambda j: (0, 0)),
            pl.BlockSpec((D, TV), lambda j: (0, j)),
            pl.BlockSpec((T, 1), lambda j: (0, 0)),
        ],
        out_specs=pl.BlockSpec((T, 1), lambda j: (0, 0)),
        scratch_shapes=[
            pltpu.VMEM((T, 2), jnp.float32),
            pltpu.VMEM((T, 2), jnp.float32),
            pltpu.SemaphoreType.DMA,
            pltpu.SemaphoreType.DMA,
        ],
        compiler_params=pltpu.CompilerParams(collective_id=0),
    )(x, W, labels2d)
    return out.reshape(T)


# baseline (device time: 196459 ns/iter reference)
import jax
import jax.numpy as jnp
from jax import lax
from jax.experimental import pallas as pl
from jax.experimental.pallas import tpu as pltpu

T = 2048
D = 4096
V_SHARD = 8192
TV = 512
NV = V_SHARD // TV


def kernel(x, W, labels):
    x = x.astype(jnp.bfloat16)
    labels2d = labels.reshape(T, 1)

    def body(x_ref, w_ref, lab_ref, out_ref,
             acc_ref, recv_ref, send_sem, recv_sem):
        j = pl.program_id(0)

        @pl.when(j == 0)
        def _init():
            acc_ref[...] = jnp.zeros_like(acc_ref)

        logits = jnp.dot(
            x_ref[...], w_ref[...].astype(jnp.bfloat16),
            preferred_element_type=jnp.float32,
        )

        s_part = jnp.sum(jnp.exp(logits), axis=1, keepdims=True)

        my_x = lax.axis_index("x")
        col0 = my_x * V_SHARD + j * TV
        cols = col0 + lax.broadcasted_iota(jnp.int32, (1, TV), 1)
        mask = lab_ref[...] == cols
        ll_part = jnp.sum(jnp.where(mask, logits, 0.0), axis=1, keepdims=True)

        acc_ref[:, 0:1] += s_part
        acc_ref[:, 1:2] += ll_part

        @pl.when(j == NV - 1)
        def _finish():
            my_y = lax.axis_index("y")
            my_z = lax.axis_index("z")
            rdma = pltpu.make_async_remote_copy(
                src_ref=acc_ref,
                dst_ref=recv_ref,
                send_sem=send_sem,
                recv_sem=recv_sem,
                device_id=(1 - my_x, my_y, my_z),
                device_id_type=pl.DeviceIdType.MESH,
            )
            rdma.start()
            rdma.wait()
            s_tot = acc_ref[:, 0:1] + recv_ref[:, 0:1]
            ll_tot = acc_ref[:, 1:2] + recv_ref[:, 1:2]
            out_ref[...] = jnp.log(s_tot) - ll_tot

    out = pl.pallas_call(
        body,
        grid=(NV,),
        out_shape=jax.ShapeDtypeStruct((T, 1), jnp.float32),
        in_specs=[
            pl.BlockSpec((T, D), lambda j: (0, 0)),
            pl.BlockSpec((D, TV), lambda j: (0, j)),
            pl.BlockSpec((T, 1), lambda j: (0, 0)),
        ],
        out_specs=pl.BlockSpec((T, 1), lambda j: (0, 0)),
        scratch_shapes=[
            pltpu.VMEM((T, 2), jnp.float32),
            pltpu.VMEM((T, 2), jnp.float32),
            pltpu.SemaphoreType.DMA,
            pltpu.SemaphoreType.DMA,
        ],
    )(x, W, labels2d)
    return out.reshape(T)


# device time: 85530 ns/iter; 2.2970x vs baseline; 2.2970x over previous
import jax
import jax.numpy as jnp
from jax import lax
from jax.experimental import pallas as pl
from jax.experimental.pallas import tpu as pltpu

T = 2048
D = 4096
V_SHARD = 8192
TV = 1024
NV = V_SHARD // TV
RB = T // 4


def kernel(x, W, labels):
    my_y = lax.axis_index("y")
    my_z = lax.axis_index("z")
    rblk = 2 * my_y + my_z
    x_rows = lax.dynamic_slice(x, (rblk * RB, 0), (RB, D))
    lab_rows = lax.dynamic_slice(
        labels.reshape(T, 1), (rblk * RB, 0), (RB, 1))

    def body(x_ref, w_ref, lab_ref, out_ref,
             acc_ref, xrecv_ref, xsend_sem, xrecv_sem, gsend_sems, grecv_sems):
        j = pl.program_id(0)

        @pl.when(j == 0)
        def _init():
            acc_ref[...] = jnp.zeros_like(acc_ref)

        logits = jnp.dot(
            x_ref[...], w_ref[...], preferred_element_type=jnp.float32,
        )

        my_x = lax.axis_index("x")
        s_part = jnp.sum(jnp.exp(logits), axis=1, keepdims=True)
        cols = (my_x * V_SHARD + j * TV
                + lax.broadcasted_iota(jnp.int32, (1, TV), 1))
        mask = lab_ref[...] == cols
        ll_part = jnp.sum(jnp.where(mask, logits, 0.0), axis=1, keepdims=True)

        acc_ref[:, 0:1] += s_part
        acc_ref[:, 1:2] += ll_part

        @pl.when(j == NV - 1)
        def _finish():
            y = lax.axis_index("y")
            z = lax.axis_index("z")
            r = 2 * y + z
            row0 = r * RB

            xr = pltpu.make_async_remote_copy(
                src_ref=acc_ref,
                dst_ref=xrecv_ref,
                send_sem=xsend_sem,
                recv_sem=xrecv_sem,
                device_id=(1 - my_x, y, z),
                device_id_type=pl.DeviceIdType.MESH,
            )
            xr.start()
            xr.wait()
            s_tot = acc_ref[:, 0:1] + xrecv_ref[:, 0:1]
            ll_tot = acc_ref[:, 1:2] + xrecv_ref[:, 1:2]
            out_ref[pl.ds(row0, RB), :] = jnp.log(s_tot) - ll_tot

            peers = [(y, 1 - z), (1 - y, z), (1 - y, 1 - z)]
            rdmas = []
            for i, (ty, tz) in enumerate(peers):
                g = pltpu.make_async_remote_copy(
                    src_ref=out_ref.at[pl.ds(row0, RB), :],
                    dst_ref=out_ref.at[pl.ds(row0, RB), :],
                    send_sem=gsend_sems.at[i],
                    recv_sem=grecv_sems.at[i],
                    device_id=(my_x, ty, tz),
                    device_id_type=pl.DeviceIdType.MESH,
                )
                g.start()
                rdmas.append(g)
            for g in rdmas:
                g.wait()

    out = pl.pallas_call(
        body,
        grid=(NV,),
        out_shape=jax.ShapeDtypeStruct((T, 1), jnp.float32),
        in_specs=[
            pl.BlockSpec((RB, D), lambda j: (0, 0)),
            pl.BlockSpec((D, TV), lambda j: (0, j)),
            pl.BlockSpec((RB, 1), lambda j: (0, 0)),
        ],
        out_specs=pl.BlockSpec((T, 1), lambda j: (0, 0)),
        scratch_shapes=[
            pltpu.VMEM((RB, 2), jnp.float32),
            pltpu.VMEM((RB, 2), jnp.float32),
            pltpu.SemaphoreType.DMA,
            pltpu.SemaphoreType.DMA,
            pltpu.SemaphoreType.DMA((3,)),
            pltpu.SemaphoreType.DMA((3,)),
        ],
        compiler_params=pltpu.CompilerParams(
            vmem_limit_bytes=100 * 1024 * 1024,
        ),
    )(x_rows, W, lab_rows)
    return out.reshape(T)


# device time: 78437 ns/iter; 2.5047x vs baseline; 1.0904x over previous
import jax
import jax.numpy as jnp
from jax import lax
from jax.experimental import pallas as pl
from jax.experimental.pallas import tpu as pltpu

T = 2048
D = 4096
V_SHARD = 8192
TV = 1024
NV = V_SHARD // TV
RB = T // 4


def kernel(x, W, labels):
    my_y = lax.axis_index("y")
    my_z = lax.axis_index("z")
    rblk = 2 * my_y + my_z
    labels2d = labels.reshape(T, 1)

    def body(rblk_ref, x_ref, w_ref, lab_ref, out_ref,
             acc_ref, xrecv_ref, xsend_sem, xrecv_sem, gsend_sems, grecv_sems):
        j = pl.program_id(0)

        @pl.when(j == 0)
        def _init():
            acc_ref[...] = jnp.zeros_like(acc_ref)

        logits = jnp.dot(
            x_ref[...], w_ref[...], preferred_element_type=jnp.float32,
        )

        my_x = lax.axis_index("x")
        s_part = jnp.sum(jnp.exp(logits), axis=1, keepdims=True)
        cols = (my_x * V_SHARD + j * TV
                + lax.broadcasted_iota(jnp.int32, (1, TV), 1))
        mask = lab_ref[...] == cols
        ll_part = jnp.sum(jnp.where(mask, logits, 0.0), axis=1, keepdims=True)

        acc_ref[:, 0:1] += s_part
        acc_ref[:, 1:2] += ll_part

        @pl.when(j == NV - 1)
        def _finish():
            y = lax.axis_index("y")
            z = lax.axis_index("z")
            r = 2 * y + z
            row0 = r * RB

            xr = pltpu.make_async_remote_copy(
                src_ref=acc_ref,
                dst_ref=xrecv_ref,
                send_sem=xsend_sem,
                recv_sem=xrecv_sem,
                device_id=(1 - my_x, y, z),
                device_id_type=pl.DeviceIdType.MESH,
            )
            xr.start()
            xr.wait()
            s_tot = acc_ref[:, 0:1] + xrecv_ref[:, 0:1]
            ll_tot = acc_ref[:, 1:2] + xrecv_ref[:, 1:2]
            out_ref[pl.ds(row0, RB), :] = jnp.log(s_tot) - ll_tot

            peers = [(y, 1 - z), (1 - y, z), (1 - y, 1 - z)]
            rdmas = []
            for i, (ty, tz) in enumerate(peers):
                g = pltpu.make_async_remote_copy(
                    src_ref=out_ref.at[pl.ds(row0, RB), :],
                    dst_ref=out_ref.at[pl.ds(row0, RB), :],
                    send_sem=gsend_sems.at[i],
                    recv_sem=grecv_sems.at[i],
                    device_id=(my_x, ty, tz),
                    device_id_type=pl.DeviceIdType.MESH,
                )
                g.start()
                rdmas.append(g)
            for g in rdmas:
                g.wait()

    grid_spec = pltpu.PrefetchScalarGridSpec(
        num_scalar_prefetch=1,
        grid=(NV,),
        in_specs=[
            pl.BlockSpec((RB, D), lambda j, rb: (rb[0], 0)),
            pl.BlockSpec((D, TV), lambda j, rb: (0, j)),
            pl.BlockSpec((RB, 1), lambda j, rb: (rb[0], 0)),
        ],
        out_specs=pl.BlockSpec((T, 1), lambda j, rb: (0, 0)),
        scratch_shapes=[
            pltpu.VMEM((RB, 2), jnp.float32),
            pltpu.VMEM((RB, 2), jnp.float32),
            pltpu.SemaphoreType.DMA,
            pltpu.SemaphoreType.DMA,
            pltpu.SemaphoreType.DMA((3,)),
            pltpu.SemaphoreType.DMA((3,)),
        ],
    )
    out = pl.pallas_call(
        body,
        grid_spec=grid_spec,
        out_shape=jax.ShapeDtypeStruct((T, 1), jnp.float32),
        compiler_params=pltpu.CompilerParams(
            vmem_limit_bytes=100 * 1024 * 1024,
        ),
    )(jnp.array([rblk], dtype=jnp.int32), x, W, labels2d)
    return out.reshape(T)


# device time: 68220 ns/iter; 2.8798x vs baseline; 1.1498x over previous
import jax
import jax.numpy as jnp
from jax import lax
from jax.experimental import pallas as pl
from jax.experimental.pallas import tpu as pltpu

T = 2048
D = 4096
V_SHARD = 8192
TV = 1024
NV = V_SHARD // TV
RB = T // 4


def kernel(x, W, labels):
    my_y = lax.axis_index("y")
    my_z = lax.axis_index("z")
    rblk = 2 * my_y + my_z
    labels2d = labels.reshape(T, 1)

    def body(rblk_ref, x_ref, w_ref, lab_ref, out_ref,
             acc_ref, xrecv_ref, xsend_sem, xrecv_sem, gsend_sems, grecv_sems):
        j = pl.program_id(0)

        @pl.when(j == 0)
        def _init():
            acc_ref[...] = jnp.zeros_like(acc_ref)
            y0 = lax.axis_index("y")
            z0 = lax.axis_index("z")
            x0 = lax.axis_index("x")
            barrier_sem = pltpu.get_barrier_semaphore()
            for dev in [(1 - x0, y0, z0), (x0, y0, 1 - z0),
                        (x0, 1 - y0, z0), (x0, 1 - y0, 1 - z0)]:
                pl.semaphore_signal(
                    barrier_sem, inc=1,
                    device_id=dev, device_id_type=pl.DeviceIdType.MESH,
                )
            pl.semaphore_wait(barrier_sem, 4)

        logits = jnp.dot(
            x_ref[...], w_ref[...], preferred_element_type=jnp.float32,
        )

        my_x = lax.axis_index("x")
        s_part = jnp.sum(jnp.exp(logits), axis=1, keepdims=True)
        cols = (my_x * V_SHARD + j * TV
                + lax.broadcasted_iota(jnp.int32, (1, TV), 1))
        mask = lab_ref[...] == cols
        ll_part = jnp.sum(jnp.where(mask, logits, 0.0), axis=1, keepdims=True)

        acc_ref[:, 0:1] += s_part
        acc_ref[:, 1:2] += ll_part

        @pl.when(j == NV - 1)
        def _finish():
            y = lax.axis_index("y")
            z = lax.axis_index("z")
            r = 2 * y + z
            row0 = r * RB

            xr = pltpu.make_async_remote_copy(
                src_ref=acc_ref,
                dst_ref=xrecv_ref,
                send_sem=xsend_sem,
                recv_sem=xrecv_sem,
                device_id=(1 - my_x, y, z),
                device_id_type=pl.DeviceIdType.MESH,
            )
            xr.start()
            xr.wait()
            s_tot = acc_ref[:, 0:1] + xrecv_ref[:, 0:1]
            ll_tot = acc_ref[:, 1:2] + xrecv_ref[:, 1:2]
            out_ref[pl.ds(row0, RB)] = (jnp.log(s_tot) - ll_tot)[:, 0]

            peers = [(y, 1 - z), (1 - y, z), (1 - y, 1 - z)]
            rdmas = []
            for i, (ty, tz) in enumerate(peers):
                g = pltpu.make_async_remote_copy(
                    src_ref=out_ref.at[pl.ds(row0, RB)],
                    dst_ref=out_ref.at[pl.ds(row0, RB)],
                    send_sem=gsend_sems.at[i],
                    recv_sem=grecv_sems.at[i],
                    device_id=(my_x, ty, tz),
                    device_id_type=pl.DeviceIdType.MESH,
                )
                g.start()
                rdmas.append(g)
            for g in rdmas:
                g.wait()

    grid_spec = pltpu.PrefetchScalarGridSpec(
        num_scalar_prefetch=1,
        grid=(NV,),
        in_specs=[
            pl.BlockSpec((RB, D), lambda j, rb: (rb[0], 0)),
            pl.BlockSpec((D, TV), lambda j, rb: (0, j)),
            pl.BlockSpec((RB, 1), lambda j, rb: (rb[0], 0)),
        ],
        out_specs=pl.BlockSpec((T,), lambda j, rb: (0,)),
        scratch_shapes=[
            pltpu.VMEM((RB, 2), jnp.float32),
            pltpu.VMEM((RB, 2), jnp.float32),
            pltpu.SemaphoreType.DMA,
            pltpu.SemaphoreType.DMA,
            pltpu.SemaphoreType.DMA((3,)),
            pltpu.SemaphoreType.DMA((3,)),
        ],
    )
    out = pl.pallas_call(
        body,
        grid_spec=grid_spec,
        out_shape=jax.ShapeDtypeStruct((T,), jnp.float32),
        compiler_params=pltpu.CompilerParams(
            vmem_limit_bytes=100 * 1024 * 1024,
            collective_id=0,
        ),
    )(jnp.array([rblk], dtype=jnp.int32), x, W, labels2d)
    return out
